# baseline (speedup 1.0000x reference)
import functools

import jax
import jax.numpy as jnp
from jax import lax
from jax.experimental import pallas as pl
from jax.experimental.pallas import tpu as pltpu
from jax.experimental.pallas import tpu_sc as plsc


def kernel(class_labels, table):
    (B,) = class_labels.shape
    V, D = table.shape

    info = plsc.get_sparse_core_info()
    NC, NS = info.num_cores, info.num_subcores
    NW = NC * NS
    b_per_w = B // NW
    assert B % (8 * NW) == 0

    CB = 128
    C = b_per_w // CB
    assert b_per_w % CB == 0

    idx = class_labels.astype(jnp.int32).reshape(NW, C, CB)

    mesh = plsc.VectorSubcoreMesh(core_axis_name="c", subcore_axis_name="s")

    @functools.partial(
        pl.kernel,
        mesh=mesh,
        out_type=jax.ShapeDtypeStruct((B, D), jnp.float32),
        scratch_types=[
            pltpu.VMEM((C, CB), jnp.int32),
            pltpu.VMEM((CB, D), jnp.float32),
            pltpu.VMEM((CB, D), jnp.float32),
            pltpu.SemaphoreType.DMA,
            pltpu.SemaphoreType.DMA,
        ],
    )
    def emb(table_hbm, idx_hbm, out_hbm, idx_v, rows0, rows1, gsem, osem):
        wid = lax.axis_index("s") * NC + lax.axis_index("c")
        base = wid * b_per_w
        pltpu.sync_copy(idx_hbm.at[wid], idx_v)

        bufs = (rows0, rows1)
        gathers = [None] * C
        outs = [None] * C
        gathers[0] = pltpu.async_copy(table_hbm.at[idx_v.at[0]], bufs[0], gsem)
        for g in range(C):
            gathers[g].wait()
            if g + 1 < C:
                if g >= 1:
                    outs[g - 1].wait()
                gathers[g + 1] = pltpu.async_copy(
                    table_hbm.at[idx_v.at[g + 1]], bufs[(g + 1) % 2], gsem
                )
            outs[g] = pltpu.async_copy(
                bufs[g % 2], out_hbm.at[pl.ds(base + g * CB, CB)], osem
            )
        for g in range(max(0, C - 2), C):
            outs[g].wait()

    return emb(table, idx)


# speedup vs baseline: 1.0487x; 1.0487x over previous
import functools

import jax
import jax.numpy as jnp
from jax import lax
from jax.experimental import pallas as pl
from jax.experimental.pallas import tpu as pltpu
from jax.experimental.pallas import tpu_sc as plsc


def kernel(class_labels, table):
    (B,) = class_labels.shape
    V, D = table.shape
    idx = class_labels if class_labels.dtype == jnp.int32 else class_labels.astype(jnp.int32)

    info = plsc.get_sparse_core_info()
    NC, NS = info.num_cores, info.num_subcores
    NW = NC * NS
    b_per_w = B // NW
    assert B % (8 * NW) == 0

    mesh = plsc.VectorSubcoreMesh(core_axis_name="c", subcore_axis_name="s")

    @functools.partial(
        pl.kernel,
        mesh=mesh,
        out_type=jax.ShapeDtypeStruct((B, D), jnp.float32),
        scratch_types=[
            pltpu.VMEM((b_per_w,), jnp.int32),
            pltpu.VMEM((b_per_w, D), jnp.float32),
            pltpu.SemaphoreType.DMA,
        ],
    )
    def emb(table_hbm, idx_hbm, out_hbm, idx_v, rows_v, sem):
        wid = lax.axis_index("s") * NC + lax.axis_index("c")
        base = wid * b_per_w
        pltpu.sync_copy(idx_hbm.at[pl.ds(base, b_per_w)], idx_v)
        pltpu.async_copy(table_hbm.at[idx_v], rows_v, sem).wait()
        pltpu.sync_copy(rows_v, out_hbm.at[pl.ds(base, b_per_w)])

    return emb(table, idx)


# speedup vs baseline: 1.1434x; 1.0904x over previous
import functools

import jax
import jax.numpy as jnp
from jax import lax
from jax.experimental import pallas as pl
from jax.experimental.pallas import tpu as pltpu
from jax.experimental.pallas import tpu_sc as plsc


def kernel(class_labels, table):
    (B,) = class_labels.shape
    V, D = table.shape
    idx = class_labels if class_labels.dtype == jnp.int32 else class_labels.astype(jnp.int32)

    info = plsc.get_sparse_core_info()
    NC, NS = info.num_cores, info.num_subcores
    NW = NC * NS
    b_per_w = B // NW
    assert B % (8 * NW) == 0

    mesh = plsc.VectorSubcoreMesh(core_axis_name="c", subcore_axis_name="s")

    @functools.partial(
        pl.kernel,
        mesh=mesh,
        out_type=jax.ShapeDtypeStruct((B, D), jnp.float32),
        scratch_types=[
            pltpu.VMEM((b_per_w,), jnp.int32),
            pltpu.VMEM((b_per_w, D), jnp.float32),
            pltpu.VMEM_SHARED((V, D), jnp.float32),
            pltpu.SemaphoreType.DMA,
        ],
    )
    def emb(table_hbm, idx_hbm, out_hbm, idx_v, rows_v, table_sp, sem):
        sid = lax.axis_index("s")
        wid = sid * NC + lax.axis_index("c")
        base = wid * b_per_w
        @pl.when(sid == 0)
        def _():
            pltpu.sync_copy(table_hbm, table_sp)
        pltpu.sync_copy(idx_hbm.at[pl.ds(base, b_per_w)], idx_v)
        plsc.subcore_barrier()
        pltpu.async_copy(table_sp.at[idx_v], rows_v, sem).wait()
        pltpu.sync_copy(rows_v, out_hbm.at[pl.ds(base, b_per_w)])

    return emb(table, idx)


# speedup vs baseline: 1.1658x; 1.0196x over previous
import functools

import jax
import jax.numpy as jnp
from jax import lax
from jax.experimental import pallas as pl
from jax.experimental.pallas import tpu as pltpu
from jax.experimental.pallas import tpu_sc as plsc


def kernel(class_labels, table):
    (B,) = class_labels.shape
    V, D = table.shape
    idx = class_labels if class_labels.dtype == jnp.int32 else class_labels.astype(jnp.int32)

    info = plsc.get_sparse_core_info()
    NC, NS = info.num_cores, info.num_subcores
    NW = NC * NS
    b_per_w = B // NW
    assert B % (8 * NW) == 0

    CB = 128
    C = b_per_w // CB
    assert b_per_w % CB == 0
    SCHUNK = 128
    n_full = V // SCHUNK
    rem = V - n_full * SCHUNK

    mesh = plsc.VectorSubcoreMesh(core_axis_name="c", subcore_axis_name="s")

    @functools.partial(
        pl.kernel,
        mesh=mesh,
        out_type=jax.ShapeDtypeStruct((B, D), jnp.float32),
        scratch_types=[
            pltpu.VMEM((b_per_w,), jnp.int32),
            pltpu.VMEM((CB, D), jnp.float32),
            pltpu.VMEM((CB, D), jnp.float32),
            pltpu.VMEM_SHARED((V, D), jnp.float32),
            pltpu.SemaphoreType.DMA,
            pltpu.SemaphoreType.DMA,
        ],
    )
    def emb(table_hbm, idx_hbm, out_hbm, idx_v, rows0, rows1, table_sp, gsem, osem):
        sid = lax.axis_index("s")
        wid = sid * NC + lax.axis_index("c")
        base = wid * b_per_w

        @pl.when(sid < n_full)
        def _():
            pltpu.sync_copy(
                table_hbm.at[pl.ds(sid * SCHUNK, SCHUNK)],
                table_sp.at[pl.ds(sid * SCHUNK, SCHUNK)],
            )

        if rem:
            @pl.when(sid == n_full)
            def _():
                pltpu.sync_copy(
                    table_hbm.at[pl.ds(n_full * SCHUNK, rem)],
                    table_sp.at[pl.ds(n_full * SCHUNK, rem)],
                )

        pltpu.sync_copy(idx_hbm.at[pl.ds(base, b_per_w)], idx_v)
        plsc.subcore_barrier()

        bufs = (rows0, rows1)
        gathers = [None] * C
        outs = [None] * C
        gathers[0] = pltpu.async_copy(
            table_sp.at[idx_v.at[pl.ds(0, CB)]], bufs[0], gsem
        )
        for g in range(C):
            gathers[g].wait()
            if g + 1 < C:
                if g >= 1:
                    outs[g - 1].wait()
                gathers[g + 1] = pltpu.async_copy(
                    table_sp.at[idx_v.at[pl.ds((g + 1) * CB, CB)]],
                    bufs[(g + 1) % 2],
                    gsem,
                )
            outs[g] = pltpu.async_copy(
                bufs[g % 2], out_hbm.at[pl.ds(base + g * CB, CB)], osem
            )
        for g in range(max(0, C - 2), C):
            outs[g].wait()

    return emb(table, idx)


# speedup vs baseline: 1.2950x; 1.1109x over previous
import functools

import jax
import jax.numpy as jnp
from jax import lax
from jax.experimental import pallas as pl
from jax.experimental.pallas import tpu as pltpu


def kernel(class_labels, table):
    (B,) = class_labels.shape
    V, D = table.shape
    idx = class_labels if class_labels.dtype == jnp.int32 else class_labels.astype(jnp.int32)

    BB = 512
    G = B // BB
    idx3 = idx.reshape(G, 1, BB)

    def body(lab_ref, tab_ref, out_ref):
        lab = lab_ref[0]
        iota_v = lax.broadcasted_iota(jnp.int32, (V, BB), 0)
        onehot_t = (iota_v == lab).astype(jnp.float32)
        out_ref[...] = lax.dot_general(
            onehot_t, tab_ref[...],
            dimension_numbers=(((0,), (0,)), ((), ())),
            preferred_element_type=jnp.float32,
        )

    out = pl.pallas_call(
        body,
        grid=(G,),
        in_specs=[
            pl.BlockSpec((1, 1, BB), lambda g: (g, 0, 0)),
            pl.BlockSpec((V, D), lambda g: (0, 0)),
        ],
        out_specs=pl.BlockSpec((BB, D), lambda g: (g, 0)),
        out_shape=jax.ShapeDtypeStruct((B, D), jnp.float32),
    )(idx3, table)
    return out
